# baseline (speedup 1.0000x reference)
import jax
import jax.numpy as jnp
from jax import lax
from jax.experimental import pallas as pl
from jax.experimental.pallas import tpu as pltpu
from jax.experimental.pallas import tpu_sc as plsc

NUM_HARMONICS = 64
NUM_ENTRIES = 100000
BATCH = 16384

NC, NS, L = 2, 16, 16
NW = NC * NS
RPW = NUM_HARMONICS // NW
ECHUNK = 2048
NECHUNK = BATCH // ECHUNK


def _body(idxf_hbm, tab_t_hbm, out_t_hbm,
          row_v, idxf_v, out_v, row_sem, idx_sem, out_sems):
    wid = lax.axis_index("s") * NC + lax.axis_index("c")

    zeros = jnp.zeros((L,), jnp.int32)
    idx_cp = pltpu.async_copy(idxf_hbm, idxf_v, idx_sem)
    out_inflight = {}
    for r in range(RPW):
        c = wid * RPW + r
        pltpu.async_copy(tab_t_hbm.at[pl.ds(c, 1)], row_v, row_sem).wait()
        if r == 0:
            idx_cp.wait()
        for k in range(NECHUNK):
            s = k & 1
            key = (r, k - 2)
            if key in out_inflight:
                out_inflight.pop(key).wait()

            def group_body(g, carry, k=k, s=s):
                for u in range(2):
                    gb = pl.ds(k * ECHUNK + (2 * g + u) * L, L)
                    f = idxf_v[gb]
                    lov = jnp.minimum(f.astype(jnp.int32), NUM_ENTRIES - 2)
                    av = f - lov.astype(jnp.float32)
                    x = plsc.load_gather(row_v, [zeros, lov])
                    y = plsc.load_gather(row_v, [zeros, lov + 1])
                    out_v[s, 0, pl.ds((2 * g + u) * L, L)] = x + av * (y - x)
                return carry

            lax.fori_loop(0, ECHUNK // (2 * L), group_body, 0)
            out_inflight[(r, k)] = pltpu.async_copy(
                out_v.at[s],
                out_t_hbm.at[pl.ds(c, 1), pl.ds(k * ECHUNK, ECHUNK)],
                out_sems[s])

    for key in sorted(out_inflight):
        out_inflight[key].wait()


@jax.jit
def _run(dial_2d, table):
    idx_f = dial_2d[:, 0] * float(NUM_ENTRIES - 1)

    mapper = pl.kernel(
        _body,
        out_type=jax.ShapeDtypeStruct((NUM_HARMONICS, BATCH), jnp.float32),
        mesh=plsc.VectorSubcoreMesh(
            core_axis_name="c", subcore_axis_name="s",
            num_cores=NC, num_subcores=NS),
        compiler_params=pltpu.CompilerParams(
            use_tc_tiling_on_sc=False, needs_layout_passes=False),
        scratch_types=[
            pltpu.VMEM((1, NUM_ENTRIES), jnp.float32),
            pltpu.VMEM((BATCH,), jnp.float32),
            pltpu.VMEM((2, 1, ECHUNK), jnp.float32),
            pltpu.SemaphoreType.DMA,
            pltpu.SemaphoreType.DMA,
            [pltpu.SemaphoreType.DMA] * 2,
        ],
    )
    out_t = mapper(idx_f, table.T)
    return out_t.T


def kernel(wave_dial_normalized, table):
    return _run(wave_dial_normalized, table)


# speedup vs baseline: 1.0056x; 1.0056x over previous
import jax
import jax.numpy as jnp
from jax import lax
from jax.experimental import pallas as pl
from jax.experimental.pallas import tpu as pltpu
from jax.experimental.pallas import tpu_sc as plsc

NUM_HARMONICS = 64
NUM_ENTRIES = 100000
BATCH = 16384

NC, NS, L = 2, 16, 16
NW = NC * NS
RPW = NUM_HARMONICS // NW
ECHUNK = 4096
NECHUNK = BATCH // ECHUNK


def _body(idxf_hbm, tab_t_hbm, out_t_hbm,
          row_v, idxf_v, out_v, row_sem, idx_sem, out_sems):
    wid = lax.axis_index("s") * NC + lax.axis_index("c")

    zeros = jnp.zeros((L,), jnp.int32)
    idx_cp = pltpu.async_copy(idxf_hbm, idxf_v, idx_sem)
    out_inflight = {}
    for r in range(RPW):
        c = wid * RPW + r
        pltpu.async_copy(tab_t_hbm.at[pl.ds(c, 1)], row_v, row_sem).wait()
        if r == 0:
            idx_cp.wait()
        for k in range(NECHUNK):
            s = k & 1
            key = (r, k - 2)
            if key in out_inflight:
                out_inflight.pop(key).wait()

            def group_body(g, carry, k=k, s=s):
                for u in range(2):
                    gb = pl.ds(k * ECHUNK + (2 * g + u) * L, L)
                    f = idxf_v[gb]
                    lov = jnp.minimum(f.astype(jnp.int32), NUM_ENTRIES - 2)
                    av = f - lov.astype(jnp.float32)
                    x = plsc.load_gather(row_v, [zeros, lov])
                    y = plsc.load_gather(row_v, [zeros, lov + 1])
                    out_v[s, 0, pl.ds((2 * g + u) * L, L)] = x + av * (y - x)
                return carry

            lax.fori_loop(0, ECHUNK // (2 * L), group_body, 0)
            out_inflight[(r, k)] = pltpu.async_copy(
                out_v.at[s],
                out_t_hbm.at[pl.ds(c, 1), pl.ds(k * ECHUNK, ECHUNK)],
                out_sems[s])

    for key in sorted(out_inflight):
        out_inflight[key].wait()


@jax.jit
def _run(dial_2d, table):
    idx_f = dial_2d[:, 0] * float(NUM_ENTRIES - 1)

    mapper = pl.kernel(
        _body,
        out_type=jax.ShapeDtypeStruct((NUM_HARMONICS, BATCH), jnp.float32),
        mesh=plsc.VectorSubcoreMesh(
            core_axis_name="c", subcore_axis_name="s",
            num_cores=NC, num_subcores=NS),
        compiler_params=pltpu.CompilerParams(
            use_tc_tiling_on_sc=False, needs_layout_passes=False),
        scratch_types=[
            pltpu.VMEM((1, NUM_ENTRIES), jnp.float32),
            pltpu.VMEM((BATCH,), jnp.float32),
            pltpu.VMEM((2, 1, ECHUNK), jnp.float32),
            pltpu.SemaphoreType.DMA,
            pltpu.SemaphoreType.DMA,
            [pltpu.SemaphoreType.DMA] * 2,
        ],
    )
    out_t = mapper(idx_f, table.T)
    return out_t.T


def kernel(wave_dial_normalized, table):
    return _run(wave_dial_normalized, table)
